# baseline (speedup 1.0000x reference)
import functools

import jax
import jax.numpy as jnp
from jax.experimental import pallas as pl
from jax.experimental.pallas import tpu as pltpu


def _shift(v, d):
    if d == 0:
        return v
    if d > 0:
        return jnp.concatenate([v[d:, :], jnp.zeros((d, 1), jnp.float32)], axis=0)
    return jnp.concatenate([jnp.zeros((-d, 1), jnp.float32), v[:d, :]], axis=0)


def _eca_body(w_ref, x_ref, o_ref, *, ntaps, hw):
    x = x_ref[...]
    lanes = jax.lax.broadcasted_iota(jnp.int32, x.shape, 1)
    is_even = lanes < hw
    inv = 1.0 / hw
    e = jnp.sum(jnp.where(is_even, x, 0.0), axis=1, keepdims=True,
                dtype=jnp.float32) * inv
    o = jnp.sum(jnp.where(is_even, 0.0, x), axis=1, keepdims=True,
                dtype=jnp.float32) * inv

    pad = ntaps // 2
    ce = jnp.zeros_like(e)
    co = jnp.zeros_like(o)
    for t in range(ntaps):
        d = t - pad
        w = w_ref[t]
        if d % 2 == 0:
            ce = ce + _shift(e, d // 2) * w
        else:
            ce = ce + _shift(o, (d - 1) // 2) * w
        if d % 2 == 0:
            co = co + _shift(o, d // 2) * w
        else:
            co = co + _shift(e, (d + 1) // 2) * w
    se = jax.nn.sigmoid(ce)
    so = jax.nn.sigmoid(co)
    o_ref[...] = x * jnp.where(is_even, se, so)


def kernel(x_nchw, conv_weight):
    B, C, H, W = x_nchw.shape
    HW = H * W
    K = conv_weight.shape[0]
    x = x_nchw.reshape(B, C // 2, 2 * HW)

    out = pl.pallas_call(
        functools.partial(_eca_body, ntaps=K, hw=HW),
        out_shape=jax.ShapeDtypeStruct(x.shape, x.dtype),
        grid=(B,),
        in_specs=[
            pl.BlockSpec(memory_space=pltpu.SMEM),
            pl.BlockSpec((None, C // 2, 2 * HW), lambda b: (b, 0, 0)),
        ],
        out_specs=pl.BlockSpec((None, C // 2, 2 * HW), lambda b: (b, 0, 0)),
        compiler_params=pltpu.CompilerParams(
            dimension_semantics=("parallel",),
            vmem_limit_bytes=64 * 1024 * 1024,
        ),
    )(conv_weight.astype(jnp.float32), x)

    return out.reshape(B, C, H, W)
